# baseline (speedup 1.0000x reference)
import jax
import jax.numpy as jnp
from jax.experimental import pallas as pl
from jax.experimental.pallas import tpu as pltpu

_BB = 32


def _body(months_ref, ch_ref, pos_ref, mtab_ref, tok_ref, out_ref):
    tok = tok_ref[...]
    bb, _, t, d = tok.shape
    n = d // 4
    m = months_ref[0]
    mo = jnp.zeros((bb, t, n), jnp.float32)
    for k in range(12):
        sel = (m == k).astype(jnp.float32)[..., None]
        mo = mo + sel * mtab_ref[k, :][None, None, :]
    ch = ch_ref[pl.ds(pl.program_id(1), 1), :]
    pos = pos_ref[...]
    out_ref[..., 0:n] = tok[..., 0:n] + ch[None, :, None, :]
    out_ref[..., n:2 * n] = tok[..., n:2 * n] + pos[None, None, :, :]
    out_ref[..., 2 * n:3 * n] = tok[..., 2 * n:3 * n] + mo[:, None, :, :]
    out_ref[..., 3 * n:] = tok[..., 3 * n:]


@jax.jit
def kernel(modality_tokens, timestamps, channel_embed, pos_embed, month_tab):
    b, t, bs, d = modality_tokens.shape
    n = d // 4
    months = timestamps[:, :, 1].astype(jnp.int32).reshape(b // _BB, _BB, t)
    tok_t = jnp.transpose(modality_tokens, (0, 2, 1, 3))
    out = pl.pallas_call(
        _body,
        grid=(b // _BB, bs),
        in_specs=[
            pl.BlockSpec((1, _BB, t), lambda i, j: (i, 0, 0)),
            pl.BlockSpec((bs, n), lambda i, j: (0, 0)),
            pl.BlockSpec((t, n), lambda i, j: (0, 0)),
            pl.BlockSpec((12, n), lambda i, j: (0, 0)),
            pl.BlockSpec((_BB, 1, t, d), lambda i, j: (i, j, 0, 0)),
        ],
        out_specs=pl.BlockSpec((_BB, 1, t, d), lambda i, j: (i, j, 0, 0)),
        out_shape=jax.ShapeDtypeStruct((b, bs, t, d), jnp.float32),
        compiler_params=pltpu.CompilerParams(
            dimension_semantics=("arbitrary", "arbitrary"),
            vmem_limit_bytes=100 * 1024 * 1024,
        ),
    )(months, channel_embed, pos_embed[:t], month_tab, tok_t)
    return jnp.transpose(out, (0, 2, 1, 3))


# speedup vs baseline: 1.2421x; 1.2421x over previous
import jax
import jax.numpy as jnp
from jax.experimental import pallas as pl
from jax.experimental.pallas import tpu as pltpu

_BB = 32


def _body(months_ref, ch_ref, pos_ref, mtab_ref, tok_ref, out_ref):
    tok = tok_ref[...]
    bb, _, t, d = tok.shape
    n = d // 4
    m = months_ref[0]
    mo = jnp.zeros((bb, t, n), jnp.float32)
    for k in range(12):
        sel = (m == k).astype(jnp.float32)[..., None]
        mo = mo + sel * mtab_ref[k, :][None, None, :]
    ch = ch_ref[...]
    pos = pos_ref[...]
    out_ref[..., 0:n] = tok[..., 0:n] + ch[None, :, None, :]
    out_ref[..., n:2 * n] = tok[..., n:2 * n] + pos[None, None, :, :]
    out_ref[..., 2 * n:3 * n] = tok[..., 2 * n:3 * n] + mo[:, None, :, :]
    out_ref[..., 3 * n:] = tok[..., 3 * n:]


@jax.jit
def kernel(modality_tokens, timestamps, channel_embed, pos_embed, month_tab):
    b, t, bs, d = modality_tokens.shape
    n = d // 4
    months = timestamps[:, :, 1].astype(jnp.int32).reshape(b // _BB, _BB, t)
    tok_t = jnp.transpose(modality_tokens, (0, 2, 1, 3))
    out = pl.pallas_call(
        _body,
        grid=(b // _BB,),
        in_specs=[
            pl.BlockSpec((1, _BB, t), lambda i: (i, 0, 0)),
            pl.BlockSpec((bs, n), lambda i: (0, 0)),
            pl.BlockSpec((t, n), lambda i: (0, 0)),
            pl.BlockSpec((12, n), lambda i: (0, 0)),
            pl.BlockSpec((_BB, bs, t, d), lambda i: (i, 0, 0, 0)),
        ],
        out_specs=pl.BlockSpec((_BB, bs, t, d), lambda i: (i, 0, 0, 0)),
        out_shape=jax.ShapeDtypeStruct((b, bs, t, d), jnp.float32),
        compiler_params=pltpu.CompilerParams(
            dimension_semantics=("arbitrary",),
            vmem_limit_bytes=100 * 1024 * 1024,
        ),
    )(months, channel_embed, pos_embed[:t], month_tab, tok_t)
    return jnp.transpose(out, (0, 2, 1, 3))
